# baseline (device time: 38899 ns/iter reference)
import jax
import jax.numpy as jnp
from jax import lax
from jax.experimental import pallas as pl
from jax.experimental.pallas import tpu as pltpu

N_DEV = 16
EPS = 1e-5
SEQ_CHUNK = 1024


def kernel(x, t_emb, W_scale, W_shift):
    b, s, c = x.shape
    c_global = c * N_DEV

    def body(x_ref, t_ref, ws_ref, wsh_ref, out_ref,
             comm_ref, send_sems, recv_sems):
        my = lax.axis_index("i")

        for i in range(s // SEQ_CHUNK):
            sl = slice(i * SEQ_CHUNK, (i + 1) * SEQ_CHUNK)
            xc = x_ref[:, sl, :]
            comm_ref[0, 0:b, sl] = jnp.sum(xc, axis=-1)
            comm_ref[0, b:2 * b, sl] = jnp.sum(xc * xc, axis=-1)

        barrier = pltpu.get_barrier_semaphore()
        for d in range(1, N_DEV):
            pl.semaphore_signal(
                barrier, inc=1,
                device_id=((my + d) % N_DEV,),
                device_id_type=pl.DeviceIdType.MESH,
            )
        pl.semaphore_wait(barrier, N_DEV - 1)

        rdmas = []
        for d in range(1, N_DEV):
            rdma = pltpu.make_async_remote_copy(
                src_ref=comm_ref.at[0],
                dst_ref=comm_ref.at[d],
                send_sem=send_sems.at[d - 1],
                recv_sem=recv_sems.at[d - 1],
                device_id=((my + d) % N_DEV,),
                device_id_type=pl.DeviceIdType.MESH,
            )
            rdma.start()
            rdmas.append(rdma)

        scale = 1.0 + jnp.dot(t_ref[...], ws_ref[...],
                              preferred_element_type=jnp.float32)
        shift = jnp.dot(t_ref[...], wsh_ref[...],
                        preferred_element_type=jnp.float32)

        for rdma in rdmas:
            rdma.wait_send()
            rdma.wait_recv()

        tot = jnp.sum(comm_ref[...], axis=0)
        mean = tot[0:b, :] * (1.0 / c_global)
        ex2 = tot[b:2 * b, :] * (1.0 / c_global)
        inv = lax.rsqrt(ex2 - mean * mean + EPS)

        for i in range(s // SEQ_CHUNK):
            sl = slice(i * SEQ_CHUNK, (i + 1) * SEQ_CHUNK)
            xc = x_ref[:, sl, :]
            m = mean[:, sl]
            iv = inv[:, sl]
            out_ref[:, sl, :] = ((xc - m[:, :, None]) * iv[:, :, None]
                                 * scale[:, None, :] + shift[:, None, :])

    return pl.pallas_call(
        body,
        out_shape=jax.ShapeDtypeStruct((b, s, c), jnp.float32),
        in_specs=[pl.BlockSpec(memory_space=pltpu.VMEM)] * 4,
        out_specs=pl.BlockSpec(memory_space=pltpu.VMEM),
        scratch_shapes=[
            pltpu.VMEM((N_DEV, 2 * b, s), jnp.float32),
            pltpu.SemaphoreType.DMA((N_DEV - 1,)),
            pltpu.SemaphoreType.DMA((N_DEV - 1,)),
        ],
        compiler_params=pltpu.CompilerParams(collective_id=0),
    )(x, t_emb, W_scale, W_shift)


# device time: 36192 ns/iter; 1.0748x vs baseline; 1.0748x over previous
import jax
import jax.numpy as jnp
from jax import lax
from jax.experimental import pallas as pl
from jax.experimental.pallas import tpu as pltpu

N_DEV = 16
EPS = 1e-5
K = 2
SUB = 1024


def kernel(x, t_emb, W_scale, W_shift):
    b, s, c = x.shape
    c_global = c * N_DEV
    ch = s // K

    def body(x_ref, t_ref, ws_ref, wsh_ref, out_ref,
             comm_ref, send_sems, recv_sems):
        my = lax.axis_index("i")

        barrier = pltpu.get_barrier_semaphore()
        for d in range(1, N_DEV):
            pl.semaphore_signal(
                barrier, inc=1,
                device_id=((my + d) % N_DEV,),
                device_id_type=pl.DeviceIdType.MESH,
            )
        pl.semaphore_wait(barrier, N_DEV - 1)

        rdmas = [[] for _ in range(K)]
        for k in range(K):
            for j in range(ch // SUB):
                lo = k * ch + j * SUB
                xc = x_ref[:, lo:lo + SUB, :]
                comm_ref[k, 0, 0:b, j * SUB:(j + 1) * SUB] = jnp.sum(xc, -1)
                comm_ref[k, 0, b:2 * b, j * SUB:(j + 1) * SUB] = (
                    jnp.sum(xc * xc, -1))
            if k > 0:
                for r in rdmas[k - 1]:
                    r.wait_send()
            for d in range(1, N_DEV):
                rdma = pltpu.make_async_remote_copy(
                    src_ref=comm_ref.at[k, 0],
                    dst_ref=comm_ref.at[k, d],
                    send_sem=send_sems.at[d - 1],
                    recv_sem=recv_sems.at[k, d - 1],
                    device_id=((my + d) % N_DEV,),
                    device_id_type=pl.DeviceIdType.MESH,
                )
                rdma.start()
                rdmas[k].append(rdma)

        scale = 1.0 + jnp.dot(t_ref[...], ws_ref[...],
                              preferred_element_type=jnp.float32)
        shift = jnp.dot(t_ref[...], wsh_ref[...],
                        preferred_element_type=jnp.float32)

        for k in range(K):
            for r in rdmas[k]:
                r.wait_recv()
            tot = jnp.sum(comm_ref[k], axis=0)
            mean = tot[0:b, :] * (1.0 / c_global)
            ex2 = tot[b:2 * b, :] * (1.0 / c_global)
            inv = lax.rsqrt(ex2 - mean * mean + EPS)
            for j in range(ch // SUB):
                lo = k * ch + j * SUB
                jsl = slice(j * SUB, (j + 1) * SUB)
                xc = x_ref[:, lo:lo + SUB, :]
                m = mean[:, jsl]
                iv = inv[:, jsl]
                out_ref[:, lo:lo + SUB, :] = (
                    (xc - m[:, :, None]) * iv[:, :, None]
                    * scale[:, None, :] + shift[:, None, :])

        for r in rdmas[K - 1]:
            r.wait_send()

    return pl.pallas_call(
        body,
        out_shape=jax.ShapeDtypeStruct((b, s, c), jnp.float32),
        in_specs=[pl.BlockSpec(memory_space=pltpu.VMEM)] * 4,
        out_specs=pl.BlockSpec(memory_space=pltpu.VMEM),
        scratch_shapes=[
            pltpu.VMEM((K, N_DEV, 2 * b, ch), jnp.float32),
            pltpu.SemaphoreType.DMA((N_DEV - 1,)),
            pltpu.SemaphoreType.DMA((K, N_DEV - 1)),
        ],
        compiler_params=pltpu.CompilerParams(collective_id=0),
    )(x, t_emb, W_scale, W_shift)
